# Initial kernel scaffold; baseline (speedup 1.0000x reference)
import jax
import jax.numpy as jnp
from jax.experimental import pallas as pl


def kernel(x, edge_index, batch, W1, b1, W2, b2, Wg1, bg1, Wg2, bg2, Wg3, bg3, Wsg, bsg, Wf1, bf1, Wf2, bf2, Wc, bc, Wo, bo):
    raise NotImplementedError("write your pallas kernel here")


# speedup vs baseline: 18.5362x; 18.5362x over previous
import functools

import jax
import jax.numpy as jnp
from jax import lax
from jax.experimental import pallas as pl
from jax.experimental.pallas import tpu as pltpu
from jax.experimental.pallas import tpu_sc as plsc

N = 10000
E = 320000
D_IN = 128
LAT = 64

NC = 2
NS = 16
NW = NC * NS
EPW = E // NW
K = 125
C = EPW // K
KD = 16
CD = E // NS // KD
N2 = 10240
NPW = N2 // NW
NPT = N // NS
ZR = 125

_MESH = dict(core_axis_name="c", subcore_axis_name="s", num_cores=NC,
             num_subcores=NS)
_SC_PARAMS = pltpu.CompilerParams(use_tc_tiling_on_sc=False,
                                  needs_layout_passes=False)


def _rsqrt16(x):
    i = plsc.bitcast(x, jnp.int32)
    i = jnp.int32(0x5F3759DF) - (i >> 1)
    y = plsc.bitcast(i, jnp.float32)
    for _ in range(3):
        y = y * (1.5 - 0.5 * x * y * y)
    return y


def _deg_body(dst_hbm, dinv_hbm, dstv, hist, colbuf, accbuf, shared, sem):
    del sem
    c = lax.axis_index("c")
    s = lax.axis_index("s")
    w = c * NS + s
    pltpu.sync_copy(dst_hbm.at[s], dstv)
    zero = jnp.zeros((16,), jnp.float32)

    def zb(i, _):
        hist[pl.ds(i * 16, 16)] = zero
        return 0
    lax.fori_loop(0, N2 // 16, zb, 0)
    ones = jnp.ones((16,), jnp.float32)

    def scat(j, _):
        plsc.addupdate_scatter(hist, [dstv[j]], ones)
        return 0
    lax.fori_loop(0, CD, scat, 0)
    pltpu.sync_copy(hist, shared.at[s])
    plsc.subcore_barrier()
    for t in range(NPW // 16):
        accbuf[pl.ds(t * 16, 16)] = zero

    def red(i, _):
        pltpu.sync_copy(shared.at[i, pl.ds(w * NPW, NPW)], colbuf)
        for t in range(NPW // 16):
            sl = pl.ds(t * 16, 16)
            accbuf[sl] = accbuf[sl] + colbuf[sl]
        return 0
    lax.fori_loop(0, NS, red, 0)
    for t in range(NPW // 16):
        sl = pl.ds(t * 16, 16)
        accbuf[sl] = _rsqrt16(accbuf[sl] + 1.0)
    pltpu.sync_copy(accbuf, dinv_hbm.at[pl.ds(w * NPW, NPW)])


_deg_dinv = functools.partial(
    pl.kernel, _deg_body,
    out_type=jax.ShapeDtypeStruct((N2,), jnp.float32),
    mesh=plsc.VectorSubcoreMesh(**_MESH),
    compiler_params=_SC_PARAMS,
    scratch_types=[
        pltpu.VMEM((CD, KD), jnp.int32),
        pltpu.VMEM((N2,), jnp.float32),
        pltpu.VMEM((NPW,), jnp.float32),
        pltpu.VMEM((NPW,), jnp.float32),
        pltpu.VMEM_SHARED((NS, N2), jnp.float32),
        pltpu.SemaphoreType.DMA,
    ],
)()


def _prop_body(u_hbm, src_hbm, dst_hbm, p_hbm, srcv, dstv, gbuf, acc,
               sem, *, d):
    c = lax.axis_index("c")
    s = lax.axis_index("s")
    w = c * NS + s
    pltpu.sync_copy(src_hbm.at[w], srcv)
    pltpu.sync_copy(dst_hbm.at[w], dstv)
    zero = jnp.zeros((16,), jnp.float32)

    def zb(i, _):
        for t in range(d // 16):
            gbuf[i, pl.ds(t * 16, 16)] = zero
        return 0
    lax.fori_loop(0, ZR, zb, 0)
    for t in range(NPT // ZR):
        pltpu.sync_copy(gbuf, acc.at[pl.ds(s * NPT + t * ZR, ZR)])
    plsc.subcore_barrier()

    def chunk(j, _):
        pltpu.async_copy(u_hbm.at[srcv.at[j]], gbuf, sem).wait()
        pltpu.sync_copy(gbuf, acc.at[dstv.at[j]], add=True)
        return 0
    lax.fori_loop(0, C, chunk, 0)
    plsc.subcore_barrier()
    pltpu.sync_copy(acc.at[pl.ds(s * NPT, NPT)],
                    p_hbm.at[c, pl.ds(s * NPT, NPT)])


def _make_prop(d):
    return functools.partial(
        pl.kernel, functools.partial(_prop_body, d=d),
        out_type=jax.ShapeDtypeStruct((NC, N, d), jnp.float32),
        mesh=plsc.VectorSubcoreMesh(**_MESH),
        compiler_params=_SC_PARAMS,
        scratch_types=[
            pltpu.VMEM((C, K), jnp.int32),
            pltpu.VMEM((C, K), jnp.int32),
            pltpu.VMEM((K, d), jnp.float32),
            pltpu.VMEM_SHARED((N, d), jnp.float32),
            pltpu.SemaphoreType.DMA,
        ],
    )()


_prop128 = _make_prop(D_IN)
_prop64 = _make_prop(LAT)

_B = 2000
_G = N // _B


def _tca_body(x_ref, w_ref, dinv_ref, u_ref):
    u_ref[...] = dinv_ref[...] * jnp.dot(
        x_ref[...], w_ref[...], preferred_element_type=jnp.float32)


def _tca(x, w, dinv):
    din, dout = w.shape
    return pl.pallas_call(
        _tca_body,
        grid=(_G,),
        in_specs=[
            pl.BlockSpec((_B, din), lambda i: (i, 0)),
            pl.BlockSpec((din, dout), lambda i: (0, 0)),
            pl.BlockSpec((_B, 1), lambda i: (i, 0)),
        ],
        out_specs=pl.BlockSpec((_B, dout), lambda i: (i, 0)),
        out_shape=jax.ShapeDtypeStruct((N, dout), jnp.float32),
    )(x, w, dinv)


def _tcb_mm_body(p_ref, u_ref, dinv_ref, b_ref, w_ref, out_ref, *, relu):
    z = dinv_ref[...] * (p_ref[0] + p_ref[1] + u_ref[...]) + b_ref[...]
    if relu:
        z = jnp.maximum(z, 0.0)
    out_ref[...] = dinv_ref[...] * jnp.dot(
        z, w_ref[...], preferred_element_type=jnp.float32)


def _tcb_ew_body(p_ref, u_ref, dinv_ref, b_ref, out_ref, *, relu):
    z = dinv_ref[...] * (p_ref[0] + p_ref[1] + u_ref[...]) + b_ref[...]
    if relu:
        z = jnp.maximum(z, 0.0)
    out_ref[...] = dinv_ref[...] * z


def _tcb(p, u, dinv, b, w, relu):
    din = u.shape[1]
    if w is None:
        return pl.pallas_call(
            functools.partial(_tcb_ew_body, relu=relu),
            grid=(_G,),
            in_specs=[
                pl.BlockSpec((NC, _B, din), lambda i: (0, i, 0)),
                pl.BlockSpec((_B, din), lambda i: (i, 0)),
                pl.BlockSpec((_B, 1), lambda i: (i, 0)),
                pl.BlockSpec((1, din), lambda i: (0, 0)),
            ],
            out_specs=pl.BlockSpec((_B, din), lambda i: (i, 0)),
            out_shape=jax.ShapeDtypeStruct((N, din), jnp.float32),
        )(p, u, dinv, b)
    dout = w.shape[1]
    return pl.pallas_call(
        functools.partial(_tcb_mm_body, relu=relu),
        grid=(_G,),
        in_specs=[
            pl.BlockSpec((NC, _B, din), lambda i: (0, i, 0)),
            pl.BlockSpec((_B, din), lambda i: (i, 0)),
            pl.BlockSpec((_B, 1), lambda i: (i, 0)),
            pl.BlockSpec((1, din), lambda i: (0, 0)),
            pl.BlockSpec((din, dout), lambda i: (0, 0)),
        ],
        out_specs=pl.BlockSpec((_B, dout), lambda i: (i, 0)),
        out_shape=jax.ShapeDtypeStruct((N, dout), jnp.float32),
    )(p, u, dinv, b, w)


def _pool_body(p_ref, u_ref, dinv_ref, wsg_ref, bsg_ref, gmax_ref):
    i = pl.program_id(0)
    h = dinv_ref[...] * (p_ref[0] + p_ref[1] + u_ref[...])
    z = jnp.dot(h, wsg_ref[...], preferred_element_type=jnp.float32)
    z = z + bsg_ref[...]
    m = jnp.max(z, axis=0, keepdims=True)

    @pl.when(i == 0)
    def _():
        gmax_ref[...] = m

    @pl.when(i > 0)
    def _():
        gmax_ref[...] = jnp.maximum(gmax_ref[...], m)


def _pool(p, u, dinv, wsg, bsg):
    return pl.pallas_call(
        _pool_body,
        grid=(_G,),
        in_specs=[
            pl.BlockSpec((NC, _B, LAT), lambda i: (0, i, 0)),
            pl.BlockSpec((_B, LAT), lambda i: (i, 0)),
            pl.BlockSpec((_B, 1), lambda i: (i, 0)),
            pl.BlockSpec((LAT, LAT), lambda i: (0, 0)),
            pl.BlockSpec((1, LAT), lambda i: (0, 0)),
        ],
        out_specs=pl.BlockSpec((1, LAT), lambda i: (0, 0)),
        out_shape=jax.ShapeDtypeStruct((1, LAT), jnp.float32),
    )(p, u, dinv, wsg, bsg)


def _head_body(g_ref, wf1, bf1, wf2, bf2, wc, bc, wo, bo, cpd_ref, comb_ref):
    g = jnp.maximum(jnp.dot(g_ref[...], wf1[...],
                            preferred_element_type=jnp.float32) + bf1[...], 0.)
    g = jnp.maximum(jnp.dot(g, wf2[...],
                            preferred_element_type=jnp.float32) + bf2[...], 0.)
    cpd_ref[...] = jnp.dot(g, wc[...],
                           preferred_element_type=jnp.float32) + bc[...]
    comb_ref[...] = jnp.dot(g, wo[...],
                            preferred_element_type=jnp.float32) + bo[...]


def _head(g, wf1, bf1, wf2, bf2, wc, bc, wo, bo):
    return pl.pallas_call(
        _head_body,
        out_shape=(jax.ShapeDtypeStruct((1, 1), jnp.float32),
                   jax.ShapeDtypeStruct((1, 1), jnp.float32)),
    )(g, wf1, bf1, wf2, bf2, wc, bc, wo, bo)


def kernel(x, edge_index, batch, W1, b1, W2, b2, Wg1, bg1, Wg2, bg2, Wg3, bg3,
           Wsg, bsg, Wf1, bf1, Wf2, bf2, Wc, bc, Wo, bo):
    del batch
    ei = edge_index.astype(jnp.int32)
    src3 = ei[0].reshape(NW, C, K)
    dst3 = ei[1].reshape(NW, C, K)
    dstd = ei[1].reshape(NS, CD, KD)

    dinv_flat = _deg_dinv(dstd)
    dinv = dinv_flat[:N].reshape(N, 1)

    u = _tca(x, W1, dinv)
    p = _prop128(u, src3, dst3)
    u = _tcb(p, u, dinv, b1.reshape(1, -1), W2, relu=True)
    p = _prop64(u, src3, dst3)
    u = _tcb(p, u, dinv, b2.reshape(1, -1), Wg1, relu=False)
    p = _prop64(u, src3, dst3)
    u = _tcb(p, u, dinv, bg1.reshape(1, -1), Wg2, relu=True)
    p = _prop64(u, src3, dst3)
    u = _tcb(p, u, dinv, bg2.reshape(1, -1), Wg3, relu=True)
    p = _prop64(u, src3, dst3)
    u = _tcb(p, u, dinv, bg3.reshape(1, -1), None, relu=True)
    p = _prop64(u, src3, dst3)
    zb = jnp.zeros((1, LAT), jnp.float32)
    u = _tcb(p, u, dinv, zb, None, relu=False)
    p = _prop64(u, src3, dst3)
    u = _tcb(p, u, dinv, zb, None, relu=False)
    p = _prop64(u, src3, dst3)
    u = _tcb(p, u, dinv, zb, None, relu=False)
    p = _prop64(u, src3, dst3)

    g = _pool(p, u, dinv, Wsg, bsg.reshape(1, -1))
    cpd, comb = _head(g, Wf1, bf1.reshape(1, -1), Wf2, bf2.reshape(1, -1),
                      Wc, bc.reshape(1, -1), Wo, bo.reshape(1, -1))
    return (cpd, comb)


# speedup vs baseline: 27.7712x; 1.4982x over previous
import functools

import jax
import jax.numpy as jnp
from jax import lax
from jax.experimental import pallas as pl
from jax.experimental.pallas import tpu as pltpu
from jax.experimental.pallas import tpu_sc as plsc

N = 10000
E = 320000
D_IN = 128
LAT = 64

NC = 2
NS = 16
NW = NC * NS
EPW = E // NW
K = 125
C = EPW // K
KD = 16
CD = E // NS // KD
N2 = 10240
NPW = N2 // NW
NPT = N // NS
ZR = 125

_MESH = dict(core_axis_name="c", subcore_axis_name="s", num_cores=NC,
             num_subcores=NS)
_SC_PARAMS = pltpu.CompilerParams(use_tc_tiling_on_sc=False,
                                  needs_layout_passes=False)


_NZT = N2 // NS
_DW = 8


def _deg_body(dst_hbm, ones_hbm, zeros_hbm, degp_hbm, dstv, onesb, zbuf, acc1,
              sem):
    c = lax.axis_index("c")
    s = lax.axis_index("s")
    w = c * NS + s
    pltpu.sync_copy(dst_hbm.at[w], dstv)
    pltpu.sync_copy(ones_hbm, onesb)
    pltpu.sync_copy(zeros_hbm, zbuf)
    pltpu.sync_copy(zbuf, acc1.at[pl.ds(s * _NZT, _NZT)])
    plsc.subcore_barrier()

    def wave(j, _):
        pltpu.sync_copy(onesb, acc1.at[dstv.at[j]], add=True)
        return 0
    lax.fori_loop(0, C, wave, 0)
    plsc.subcore_barrier()
    pltpu.sync_copy(acc1.at[pl.ds(s * _NZT, _NZT)],
                    degp_hbm.at[c, pl.ds(s * _NZT, _NZT)])


_deg_call = functools.partial(
    pl.kernel, _deg_body,
    out_type=jax.ShapeDtypeStruct((NC, N2, _DW), jnp.float32),
    mesh=plsc.VectorSubcoreMesh(**_MESH),
    compiler_params=_SC_PARAMS,
    scratch_types=[
        pltpu.VMEM((C, K), jnp.int32),
        pltpu.VMEM((K, _DW), jnp.float32),
        pltpu.VMEM((_NZT, _DW), jnp.float32),
        pltpu.VMEM_SHARED((N2, _DW), jnp.float32),
        pltpu.SemaphoreType.DMA,
    ],
)()


def _prop_body(u_hbm, src_hbm, dst_hbm, p_hbm, srcv, dstv, ga, gb, acc,
               sema, semb, *, d, k, c_):
    c = lax.axis_index("c")
    s = lax.axis_index("s")
    w = c * NS + s
    pltpu.sync_copy(src_hbm.at[w], srcv)
    pltpu.sync_copy(dst_hbm.at[w], dstv)
    zero = jnp.zeros((16,), jnp.float32)

    def zb(i, _):
        for t in range(d // 16):
            gb[i, pl.ds(t * 16, 16)] = zero
        return 0
    lax.fori_loop(0, k, zb, 0)
    nfull, rem = divmod(NPT, k)
    for t in range(nfull):
        pltpu.sync_copy(gb, acc.at[pl.ds(s * NPT + t * k, k)])
    if rem:
        pltpu.sync_copy(gb.at[pl.ds(0, rem)],
                        acc.at[pl.ds(s * NPT + nfull * k, rem)])
    pltpu.async_copy(u_hbm.at[srcv.at[0]], ga, sema)
    plsc.subcore_barrier()

    def chunk2(i, _):
        j = 2 * i
        db = pltpu.async_copy(u_hbm.at[srcv.at[j + 1]], gb, semb)
        pltpu.make_async_copy(u_hbm.at[srcv.at[j]], ga, sema).wait()
        pltpu.sync_copy(ga, acc.at[dstv.at[j]], add=True)

        @pl.when(j + 2 < c_)
        def _():
            pltpu.async_copy(u_hbm.at[srcv.at[j + 2]], ga, sema)
        db.wait()
        pltpu.sync_copy(gb, acc.at[dstv.at[j + 1]], add=True)
        return 0
    lax.fori_loop(0, c_ // 2, chunk2, 0)
    plsc.subcore_barrier()
    pltpu.sync_copy(acc.at[pl.ds(s * NPT, NPT)],
                    p_hbm.at[c, pl.ds(s * NPT, NPT)])


def _make_prop(d, k):
    c_ = EPW // k
    return functools.partial(
        pl.kernel, functools.partial(_prop_body, d=d, k=k, c_=c_),
        out_type=jax.ShapeDtypeStruct((NC, N, d), jnp.float32),
        mesh=plsc.VectorSubcoreMesh(**_MESH),
        compiler_params=_SC_PARAMS,
        scratch_types=[
            pltpu.VMEM((c_, k), jnp.int32),
            pltpu.VMEM((c_, k), jnp.int32),
            pltpu.VMEM((k, d), jnp.float32),
            pltpu.VMEM((k, d), jnp.float32),
            pltpu.VMEM_SHARED((N, d), jnp.float32),
            pltpu.SemaphoreType.DMA,
            pltpu.SemaphoreType.DMA,
        ],
    )()


_prop128 = _make_prop(D_IN, 100)
_prop64 = _make_prop(LAT, K)

_B = 2000
_G = N // _B


def _tca_body(degp_ref, x_ref, w_ref, dinv_ref, u_ref):
    deg = degp_ref[0, :, 0:1] + degp_ref[1, :, 0:1] + 1.0
    dinv = jax.lax.rsqrt(deg)
    dinv_ref[...] = dinv
    u_ref[...] = dinv * jnp.dot(
        x_ref[...], w_ref[...], preferred_element_type=jnp.float32)


def _tca(degp, x, w):
    din, dout = w.shape
    return pl.pallas_call(
        _tca_body,
        grid=(_G,),
        in_specs=[
            pl.BlockSpec((NC, _B, _DW), lambda i: (0, i, 0)),
            pl.BlockSpec((_B, din), lambda i: (i, 0)),
            pl.BlockSpec((din, dout), lambda i: (0, 0)),
        ],
        out_specs=(pl.BlockSpec((_B, 1), lambda i: (i, 0)),
                   pl.BlockSpec((_B, dout), lambda i: (i, 0))),
        out_shape=(jax.ShapeDtypeStruct((N, 1), jnp.float32),
                   jax.ShapeDtypeStruct((N, dout), jnp.float32)),
    )(degp, x, w)


def _tcb_mm_body(p_ref, u_ref, dinv_ref, b_ref, w_ref, out_ref, *, relu):
    z = dinv_ref[...] * (p_ref[0] + p_ref[1] + u_ref[...]) + b_ref[...]
    if relu:
        z = jnp.maximum(z, 0.0)
    out_ref[...] = dinv_ref[...] * jnp.dot(
        z, w_ref[...], preferred_element_type=jnp.float32)


def _tcb_ew_body(p_ref, u_ref, dinv_ref, b_ref, out_ref, *, relu):
    z = dinv_ref[...] * (p_ref[0] + p_ref[1] + u_ref[...]) + b_ref[...]
    if relu:
        z = jnp.maximum(z, 0.0)
    out_ref[...] = dinv_ref[...] * z


def _tcb(p, u, dinv, b, w, relu):
    din = u.shape[1]
    if w is None:
        return pl.pallas_call(
            functools.partial(_tcb_ew_body, relu=relu),
            grid=(_G,),
            in_specs=[
                pl.BlockSpec((NC, _B, din), lambda i: (0, i, 0)),
                pl.BlockSpec((_B, din), lambda i: (i, 0)),
                pl.BlockSpec((_B, 1), lambda i: (i, 0)),
                pl.BlockSpec((1, din), lambda i: (0, 0)),
            ],
            out_specs=pl.BlockSpec((_B, din), lambda i: (i, 0)),
            out_shape=jax.ShapeDtypeStruct((N, din), jnp.float32),
        )(p, u, dinv, b)
    dout = w.shape[1]
    return pl.pallas_call(
        functools.partial(_tcb_mm_body, relu=relu),
        grid=(_G,),
        in_specs=[
            pl.BlockSpec((NC, _B, din), lambda i: (0, i, 0)),
            pl.BlockSpec((_B, din), lambda i: (i, 0)),
            pl.BlockSpec((_B, 1), lambda i: (i, 0)),
            pl.BlockSpec((1, din), lambda i: (0, 0)),
            pl.BlockSpec((din, dout), lambda i: (0, 0)),
        ],
        out_specs=pl.BlockSpec((_B, dout), lambda i: (i, 0)),
        out_shape=jax.ShapeDtypeStruct((N, dout), jnp.float32),
    )(p, u, dinv, b, w)


def _pool_body(p_ref, u_ref, dinv_ref, wsg_ref, bsg_ref, gmax_ref):
    i = pl.program_id(0)
    h = dinv_ref[...] * (p_ref[0] + p_ref[1] + u_ref[...])
    z = jnp.dot(h, wsg_ref[...], preferred_element_type=jnp.float32)
    z = z + bsg_ref[...]
    m = jnp.max(z, axis=0, keepdims=True)

    @pl.when(i == 0)
    def _():
        gmax_ref[...] = m

    @pl.when(i > 0)
    def _():
        gmax_ref[...] = jnp.maximum(gmax_ref[...], m)


def _pool(p, u, dinv, wsg, bsg):
    return pl.pallas_call(
        _pool_body,
        grid=(_G,),
        in_specs=[
            pl.BlockSpec((NC, _B, LAT), lambda i: (0, i, 0)),
            pl.BlockSpec((_B, LAT), lambda i: (i, 0)),
            pl.BlockSpec((_B, 1), lambda i: (i, 0)),
            pl.BlockSpec((LAT, LAT), lambda i: (0, 0)),
            pl.BlockSpec((1, LAT), lambda i: (0, 0)),
        ],
        out_specs=pl.BlockSpec((1, LAT), lambda i: (0, 0)),
        out_shape=jax.ShapeDtypeStruct((1, LAT), jnp.float32),
    )(p, u, dinv, wsg, bsg)


def _head_body(g_ref, wf1, bf1, wf2, bf2, wc, bc, wo, bo, cpd_ref, comb_ref):
    g = jnp.maximum(jnp.dot(g_ref[...], wf1[...],
                            preferred_element_type=jnp.float32) + bf1[...], 0.)
    g = jnp.maximum(jnp.dot(g, wf2[...],
                            preferred_element_type=jnp.float32) + bf2[...], 0.)
    cpd_ref[...] = jnp.dot(g, wc[...],
                           preferred_element_type=jnp.float32) + bc[...]
    comb_ref[...] = jnp.dot(g, wo[...],
                            preferred_element_type=jnp.float32) + bo[...]


def _head(g, wf1, bf1, wf2, bf2, wc, bc, wo, bo):
    return pl.pallas_call(
        _head_body,
        out_shape=(jax.ShapeDtypeStruct((1, 1), jnp.float32),
                   jax.ShapeDtypeStruct((1, 1), jnp.float32)),
    )(g, wf1, bf1, wf2, bf2, wc, bc, wo, bo)


def kernel(x, edge_index, batch, W1, b1, W2, b2, Wg1, bg1, Wg2, bg2, Wg3, bg3,
           Wsg, bsg, Wf1, bf1, Wf2, bf2, Wc, bc, Wo, bo):
    del batch
    ei = edge_index.astype(jnp.int32)
    src3 = ei[0].reshape(NW, C, K)
    dst3 = ei[1].reshape(NW, C, K)
    srcw = ei[0].reshape(NW, 100, 100)
    dstw = ei[1].reshape(NW, 100, 100)

    ones_col = jnp.ones((K, _DW), jnp.float32)
    zeros_col = jnp.zeros((_NZT, _DW), jnp.float32)
    degp = _deg_call(dst3, ones_col, zeros_col)

    dinv, u = _tca(degp, x, W1)
    p = _prop128(u, srcw, dstw)
    u = _tcb(p, u, dinv, b1.reshape(1, -1), W2, relu=True)
    p = _prop64(u, src3, dst3)
    u = _tcb(p, u, dinv, b2.reshape(1, -1), Wg1, relu=False)
    p = _prop64(u, src3, dst3)
    u = _tcb(p, u, dinv, bg1.reshape(1, -1), Wg2, relu=True)
    p = _prop64(u, src3, dst3)
    u = _tcb(p, u, dinv, bg2.reshape(1, -1), Wg3, relu=True)
    p = _prop64(u, src3, dst3)
    u = _tcb(p, u, dinv, bg3.reshape(1, -1), None, relu=True)
    p = _prop64(u, src3, dst3)
    zb = jnp.zeros((1, LAT), jnp.float32)
    u = _tcb(p, u, dinv, zb, None, relu=False)
    p = _prop64(u, src3, dst3)
    u = _tcb(p, u, dinv, zb, None, relu=False)
    p = _prop64(u, src3, dst3)
    u = _tcb(p, u, dinv, zb, None, relu=False)
    p = _prop64(u, src3, dst3)

    g = _pool(p, u, dinv, Wsg, bsg.reshape(1, -1))
    cpd, comb = _head(g, Wf1, bf1.reshape(1, -1), Wf2, bf2.reshape(1, -1),
                      Wc, bc.reshape(1, -1), Wo, bo.reshape(1, -1))
    return (cpd, comb)


# speedup vs baseline: 28.3001x; 1.0190x over previous
import functools

import jax
import jax.numpy as jnp
from jax import lax
from jax.experimental import pallas as pl
from jax.experimental.pallas import tpu as pltpu
from jax.experimental.pallas import tpu_sc as plsc

N = 10000
E = 320000
D_IN = 128
LAT = 64

NC = 2
NS = 16
NW = NC * NS
EPW = E // NW
K = 125
C = EPW // K
KD = 16
CD = E // NS // KD
N2 = 10240
NPW = N2 // NW
NPT = N // NS
ZR = 125

_MESH = dict(core_axis_name="c", subcore_axis_name="s", num_cores=NC,
             num_subcores=NS)
_SC_PARAMS = pltpu.CompilerParams(use_tc_tiling_on_sc=False,
                                  needs_layout_passes=False)


_NZT = N2 // NS
_DW = 8


def _deg_body(dst_hbm, ones_hbm, zeros_hbm, degp_hbm, dstv, onesb, zbuf, acc1,
              sem):
    c = lax.axis_index("c")
    s = lax.axis_index("s")
    w = c * NS + s
    pltpu.sync_copy(dst_hbm.at[w], dstv)
    pltpu.sync_copy(ones_hbm, onesb)
    pltpu.sync_copy(zeros_hbm, zbuf)
    pltpu.sync_copy(zbuf, acc1.at[pl.ds(s * _NZT, _NZT)])
    plsc.subcore_barrier()

    def wave(i, _):
        dds = [pltpu.async_copy(onesb, acc1.at[dstv.at[i * 8 + t]], sem,
                                add=True)
               for t in range(8)]
        for dd in dds:
            dd.wait()
        return 0
    lax.fori_loop(0, C // 8, wave, 0)
    plsc.subcore_barrier()
    pltpu.sync_copy(acc1.at[pl.ds(s * _NZT, _NZT)],
                    degp_hbm.at[c, pl.ds(s * _NZT, _NZT)])


_deg_call = functools.partial(
    pl.kernel, _deg_body,
    out_type=jax.ShapeDtypeStruct((NC, N2, _DW), jnp.float32),
    mesh=plsc.VectorSubcoreMesh(**_MESH),
    compiler_params=_SC_PARAMS,
    scratch_types=[
        pltpu.VMEM((C, K), jnp.int32),
        pltpu.VMEM((K, _DW), jnp.float32),
        pltpu.VMEM((_NZT, _DW), jnp.float32),
        pltpu.VMEM_SHARED((N2, _DW), jnp.float32),
        pltpu.SemaphoreType.DMA,
    ],
)()


def _prop_body(u_hbm, src_hbm, dst_hbm, p_hbm, *rest, d, k, c_, nb):
    srcv, dstv = rest[0], rest[1]
    gbufs = rest[2:2 + nb]
    gsems = rest[2 + nb:2 + 2 * nb]
    ssems = rest[2 + 2 * nb:2 + 3 * nb]
    acc = rest[2 + 3 * nb]
    c = lax.axis_index("c")
    s = lax.axis_index("s")
    w = c * NS + s
    pltpu.sync_copy(src_hbm.at[w], srcv)
    pltpu.sync_copy(dst_hbm.at[w], dstv)
    zero = jnp.zeros((16,), jnp.float32)

    def zb(i, _):
        for t in range(d // 16):
            gbufs[0][i, pl.ds(t * 16, 16)] = zero
        return 0
    lax.fori_loop(0, k, zb, 0)
    nfull, rem = divmod(NPT, k)
    for t in range(nfull):
        pltpu.sync_copy(gbufs[0], acc.at[pl.ds(s * NPT + t * k, k)])
    if rem:
        pltpu.sync_copy(gbufs[0].at[pl.ds(0, rem)],
                        acc.at[pl.ds(s * NPT + nfull * k, rem)])
    for t in range(nb):
        pltpu.async_copy(u_hbm.at[srcv.at[t]], gbufs[t], gsems[t])
    plsc.subcore_barrier()

    def ring(i, _):
        base = i * nb
        for t in range(nb):
            j = base + t
            pltpu.make_async_copy(u_hbm.at[srcv.at[j]], gbufs[t],
                                  gsems[t]).wait()
            pltpu.async_copy(gbufs[t], acc.at[dstv.at[j]], ssems[t],
                             add=True)
        for t in range(nb):
            j = base + t
            pltpu.make_async_copy(gbufs[t], acc.at[dstv.at[j]],
                                  ssems[t]).wait()

            @pl.when(j + nb < c_)
            def _():
                pltpu.async_copy(u_hbm.at[srcv.at[j + nb]], gbufs[t],
                                 gsems[t])
        return 0
    lax.fori_loop(0, c_ // nb, ring, 0)
    plsc.subcore_barrier()
    pltpu.sync_copy(acc.at[pl.ds(s * NPT, NPT)],
                    p_hbm.at[c, pl.ds(s * NPT, NPT)])


def _make_prop(d, k, nb):
    c_ = EPW // k
    assert c_ % nb == 0
    return functools.partial(
        pl.kernel, functools.partial(_prop_body, d=d, k=k, c_=c_, nb=nb),
        out_type=jax.ShapeDtypeStruct((NC, N, d), jnp.float32),
        mesh=plsc.VectorSubcoreMesh(**_MESH),
        compiler_params=_SC_PARAMS,
        scratch_types=(
            [pltpu.VMEM((c_, k), jnp.int32),
             pltpu.VMEM((c_, k), jnp.int32)]
            + [pltpu.VMEM((k, d), jnp.float32)] * nb
            + [pltpu.SemaphoreType.DMA] * (2 * nb)
            + [pltpu.VMEM_SHARED((N, d), jnp.float32)]
        ),
    )()


_prop128 = _make_prop(D_IN, 100, 2)
_prop64 = _make_prop(LAT, K, 4)

_B = 2000
_G = N // _B


def _tca_body(degp_ref, x_ref, w_ref, dinv_ref, u_ref):
    deg = degp_ref[0, :, 0:1] + degp_ref[1, :, 0:1] + 1.0
    dinv = jax.lax.rsqrt(deg)
    dinv_ref[...] = dinv
    u_ref[...] = dinv * jnp.dot(
        x_ref[...], w_ref[...], preferred_element_type=jnp.float32)


def _tca(degp, x, w):
    din, dout = w.shape
    return pl.pallas_call(
        _tca_body,
        grid=(_G,),
        in_specs=[
            pl.BlockSpec((NC, _B, _DW), lambda i: (0, i, 0)),
            pl.BlockSpec((_B, din), lambda i: (i, 0)),
            pl.BlockSpec((din, dout), lambda i: (0, 0)),
        ],
        out_specs=(pl.BlockSpec((_B, 1), lambda i: (i, 0)),
                   pl.BlockSpec((_B, dout), lambda i: (i, 0))),
        out_shape=(jax.ShapeDtypeStruct((N, 1), jnp.float32),
                   jax.ShapeDtypeStruct((N, dout), jnp.float32)),
    )(degp, x, w)


def _tcb_mm_body(p_ref, u_ref, dinv_ref, b_ref, w_ref, out_ref, *, relu):
    z = dinv_ref[...] * (p_ref[0] + p_ref[1] + u_ref[...]) + b_ref[...]
    if relu:
        z = jnp.maximum(z, 0.0)
    out_ref[...] = dinv_ref[...] * jnp.dot(
        z, w_ref[...], preferred_element_type=jnp.float32)


def _tcb_ew_body(p_ref, u_ref, dinv_ref, b_ref, out_ref, *, relu):
    z = dinv_ref[...] * (p_ref[0] + p_ref[1] + u_ref[...]) + b_ref[...]
    if relu:
        z = jnp.maximum(z, 0.0)
    out_ref[...] = dinv_ref[...] * z


def _tcb(p, u, dinv, b, w, relu):
    din = u.shape[1]
    if w is None:
        return pl.pallas_call(
            functools.partial(_tcb_ew_body, relu=relu),
            grid=(_G,),
            in_specs=[
                pl.BlockSpec((NC, _B, din), lambda i: (0, i, 0)),
                pl.BlockSpec((_B, din), lambda i: (i, 0)),
                pl.BlockSpec((_B, 1), lambda i: (i, 0)),
                pl.BlockSpec((1, din), lambda i: (0, 0)),
            ],
            out_specs=pl.BlockSpec((_B, din), lambda i: (i, 0)),
            out_shape=jax.ShapeDtypeStruct((N, din), jnp.float32),
        )(p, u, dinv, b)
    dout = w.shape[1]
    return pl.pallas_call(
        functools.partial(_tcb_mm_body, relu=relu),
        grid=(_G,),
        in_specs=[
            pl.BlockSpec((NC, _B, din), lambda i: (0, i, 0)),
            pl.BlockSpec((_B, din), lambda i: (i, 0)),
            pl.BlockSpec((_B, 1), lambda i: (i, 0)),
            pl.BlockSpec((1, din), lambda i: (0, 0)),
            pl.BlockSpec((din, dout), lambda i: (0, 0)),
        ],
        out_specs=pl.BlockSpec((_B, dout), lambda i: (i, 0)),
        out_shape=jax.ShapeDtypeStruct((N, dout), jnp.float32),
    )(p, u, dinv, b, w)


def _pool_body(p_ref, u_ref, dinv_ref, wsg_ref, bsg_ref, gmax_ref):
    i = pl.program_id(0)
    h = dinv_ref[...] * (p_ref[0] + p_ref[1] + u_ref[...])
    z = jnp.dot(h, wsg_ref[...], preferred_element_type=jnp.float32)
    z = z + bsg_ref[...]
    m = jnp.max(z, axis=0, keepdims=True)

    @pl.when(i == 0)
    def _():
        gmax_ref[...] = m

    @pl.when(i > 0)
    def _():
        gmax_ref[...] = jnp.maximum(gmax_ref[...], m)


def _pool(p, u, dinv, wsg, bsg):
    return pl.pallas_call(
        _pool_body,
        grid=(_G,),
        in_specs=[
            pl.BlockSpec((NC, _B, LAT), lambda i: (0, i, 0)),
            pl.BlockSpec((_B, LAT), lambda i: (i, 0)),
            pl.BlockSpec((_B, 1), lambda i: (i, 0)),
            pl.BlockSpec((LAT, LAT), lambda i: (0, 0)),
            pl.BlockSpec((1, LAT), lambda i: (0, 0)),
        ],
        out_specs=pl.BlockSpec((1, LAT), lambda i: (0, 0)),
        out_shape=jax.ShapeDtypeStruct((1, LAT), jnp.float32),
    )(p, u, dinv, wsg, bsg)


def _head_body(g_ref, wf1, bf1, wf2, bf2, wc, bc, wo, bo, cpd_ref, comb_ref):
    g = jnp.maximum(jnp.dot(g_ref[...], wf1[...],
                            preferred_element_type=jnp.float32) + bf1[...], 0.)
    g = jnp.maximum(jnp.dot(g, wf2[...],
                            preferred_element_type=jnp.float32) + bf2[...], 0.)
    cpd_ref[...] = jnp.dot(g, wc[...],
                           preferred_element_type=jnp.float32) + bc[...]
    comb_ref[...] = jnp.dot(g, wo[...],
                            preferred_element_type=jnp.float32) + bo[...]


def _head(g, wf1, bf1, wf2, bf2, wc, bc, wo, bo):
    return pl.pallas_call(
        _head_body,
        out_shape=(jax.ShapeDtypeStruct((1, 1), jnp.float32),
                   jax.ShapeDtypeStruct((1, 1), jnp.float32)),
    )(g, wf1, bf1, wf2, bf2, wc, bc, wo, bo)


def kernel(x, edge_index, batch, W1, b1, W2, b2, Wg1, bg1, Wg2, bg2, Wg3, bg3,
           Wsg, bsg, Wf1, bf1, Wf2, bf2, Wc, bc, Wo, bo):
    del batch
    ei = edge_index.astype(jnp.int32)
    src3 = ei[0].reshape(NW, C, K)
    dst3 = ei[1].reshape(NW, C, K)
    srcw = ei[0].reshape(NW, 100, 100)
    dstw = ei[1].reshape(NW, 100, 100)

    ones_col = jnp.ones((K, _DW), jnp.float32)
    zeros_col = jnp.zeros((_NZT, _DW), jnp.float32)
    degp = _deg_call(dst3, ones_col, zeros_col)

    dinv, u = _tca(degp, x, W1)
    p = _prop128(u, srcw, dstw)
    u = _tcb(p, u, dinv, b1.reshape(1, -1), W2, relu=True)
    p = _prop64(u, src3, dst3)
    u = _tcb(p, u, dinv, b2.reshape(1, -1), Wg1, relu=False)
    p = _prop64(u, src3, dst3)
    u = _tcb(p, u, dinv, bg1.reshape(1, -1), Wg2, relu=True)
    p = _prop64(u, src3, dst3)
    u = _tcb(p, u, dinv, bg2.reshape(1, -1), Wg3, relu=True)
    p = _prop64(u, src3, dst3)
    u = _tcb(p, u, dinv, bg3.reshape(1, -1), None, relu=True)
    p = _prop64(u, src3, dst3)
    zb = jnp.zeros((1, LAT), jnp.float32)
    u = _tcb(p, u, dinv, zb, None, relu=False)
    p = _prop64(u, src3, dst3)
    u = _tcb(p, u, dinv, zb, None, relu=False)
    p = _prop64(u, src3, dst3)
    u = _tcb(p, u, dinv, zb, None, relu=False)
    p = _prop64(u, src3, dst3)

    g = _pool(p, u, dinv, Wsg, bsg.reshape(1, -1))
    cpd, comb = _head(g, Wf1, bf1.reshape(1, -1), Wf2, bf2.reshape(1, -1),
                      Wc, bc.reshape(1, -1), Wo, bo.reshape(1, -1))
    return (cpd, comb)


# speedup vs baseline: 29.0070x; 1.0250x over previous
import functools

import jax
import jax.numpy as jnp
from jax import lax
from jax.experimental import pallas as pl
from jax.experimental.pallas import tpu as pltpu
from jax.experimental.pallas import tpu_sc as plsc

N = 10000
E = 320000
D_IN = 128
LAT = 64

NC = 2
NS = 16
NW = NC * NS
EPW = E // NW
K = 125
C = EPW // K
KD = 16
CD = E // NS // KD
N2 = 10240
NPW = N2 // NW
NPT = N // NS
ZR = 125

_MESH = dict(core_axis_name="c", subcore_axis_name="s", num_cores=NC,
             num_subcores=NS)
_SC_PARAMS = pltpu.CompilerParams(use_tc_tiling_on_sc=False,
                                  needs_layout_passes=False)


_NZT = N2 // NS
_DW = 8


def _deg_body(dst_hbm, ones_hbm, zeros_hbm, degp_hbm, dstv, onesb, zbuf, acc1,
              sem):
    c = lax.axis_index("c")
    s = lax.axis_index("s")
    w = c * NS + s
    pltpu.sync_copy(dst_hbm.at[w], dstv)
    pltpu.sync_copy(ones_hbm, onesb)
    pltpu.sync_copy(zeros_hbm, zbuf)
    pltpu.sync_copy(zbuf, acc1.at[pl.ds(s * _NZT, _NZT)])
    plsc.subcore_barrier()

    def wave(j, _):
        pltpu.sync_copy(onesb, acc1.at[dstv.at[j]], add=True)
        return 0
    lax.fori_loop(0, C, wave, 0)
    plsc.subcore_barrier()
    pltpu.sync_copy(acc1.at[pl.ds(s * _NZT, _NZT)],
                    degp_hbm.at[c, pl.ds(s * _NZT, _NZT)])


_deg_call = functools.partial(
    pl.kernel, _deg_body,
    out_type=jax.ShapeDtypeStruct((NC, N2, _DW), jnp.float32),
    mesh=plsc.VectorSubcoreMesh(**_MESH),
    compiler_params=_SC_PARAMS,
    scratch_types=[
        pltpu.VMEM((C, K), jnp.int32),
        pltpu.VMEM((K, _DW), jnp.float32),
        pltpu.VMEM((_NZT, _DW), jnp.float32),
        pltpu.VMEM_SHARED((N2, _DW), jnp.float32),
        pltpu.SemaphoreType.DMA,
    ],
)()


def _prop_body(u_hbm, src_hbm, dst_hbm, p_hbm, *rest, d, k, c_, nb):
    srcv, dstv = rest[0], rest[1]
    gbufs = rest[2:2 + nb]
    gsems = rest[2 + nb:2 + 2 * nb]
    ssems = rest[2 + 2 * nb:2 + 3 * nb]
    acc = rest[2 + 3 * nb]
    c = lax.axis_index("c")
    s = lax.axis_index("s")
    w = c * NS + s
    pltpu.sync_copy(src_hbm.at[w], srcv)
    pltpu.sync_copy(dst_hbm.at[w], dstv)
    zero = jnp.zeros((16,), jnp.float32)

    def zb(i, _):
        for t in range(d // 16):
            gbufs[0][i, pl.ds(t * 16, 16)] = zero
        return 0
    lax.fori_loop(0, k, zb, 0)
    nfull, rem = divmod(NPT, k)
    for t in range(nfull):
        pltpu.sync_copy(gbufs[0], acc.at[pl.ds(s * NPT + t * k, k)])
    if rem:
        pltpu.sync_copy(gbufs[0].at[pl.ds(0, rem)],
                        acc.at[pl.ds(s * NPT + nfull * k, rem)])
    for t in range(nb):
        pltpu.async_copy(u_hbm.at[srcv.at[t]], gbufs[t], gsems[t])
    plsc.subcore_barrier()

    def ring(i, _):
        base = i * nb
        for t in range(nb):
            j = base + t
            pltpu.make_async_copy(u_hbm.at[srcv.at[j]], gbufs[t],
                                  gsems[t]).wait()
            pltpu.async_copy(gbufs[t], acc.at[dstv.at[j]], ssems[t],
                             add=True)
        for t in range(nb):
            j = base + t
            pltpu.make_async_copy(gbufs[t], acc.at[dstv.at[j]],
                                  ssems[t]).wait()

            @pl.when(j + nb < c_)
            def _():
                pltpu.async_copy(u_hbm.at[srcv.at[j + nb]], gbufs[t],
                                 gsems[t])
        return 0
    lax.fori_loop(0, c_ // nb, ring, 0)
    plsc.subcore_barrier()
    pltpu.sync_copy(acc.at[pl.ds(s * NPT, NPT)],
                    p_hbm.at[c, pl.ds(s * NPT, NPT)])


def _make_prop(d, k, nb):
    c_ = EPW // k
    assert c_ % nb == 0
    return functools.partial(
        pl.kernel, functools.partial(_prop_body, d=d, k=k, c_=c_, nb=nb),
        out_type=jax.ShapeDtypeStruct((NC, N, d), jnp.float32),
        mesh=plsc.VectorSubcoreMesh(**_MESH),
        compiler_params=_SC_PARAMS,
        scratch_types=(
            [pltpu.VMEM((c_, k), jnp.int32),
             pltpu.VMEM((c_, k), jnp.int32)]
            + [pltpu.VMEM((k, d), jnp.float32)] * nb
            + [pltpu.SemaphoreType.DMA] * (2 * nb)
            + [pltpu.VMEM_SHARED((N, d), jnp.float32)]
        ),
    )()


_prop128 = _make_prop(D_IN, 50, 4)
_prop64 = _make_prop(LAT, K, 4)

_B = 2000
_G = N // _B


def _tca_body(degp_ref, x_ref, w_ref, dinv_ref, u_ref):
    deg = degp_ref[0, :, 0:1] + degp_ref[1, :, 0:1] + 1.0
    dinv = jax.lax.rsqrt(deg)
    dinv_ref[...] = dinv
    u_ref[...] = dinv * jnp.dot(
        x_ref[...], w_ref[...], preferred_element_type=jnp.float32)


def _tca(degp, x, w):
    din, dout = w.shape
    return pl.pallas_call(
        _tca_body,
        grid=(_G,),
        in_specs=[
            pl.BlockSpec((NC, _B, _DW), lambda i: (0, i, 0)),
            pl.BlockSpec((_B, din), lambda i: (i, 0)),
            pl.BlockSpec((din, dout), lambda i: (0, 0)),
        ],
        out_specs=(pl.BlockSpec((_B, 1), lambda i: (i, 0)),
                   pl.BlockSpec((_B, dout), lambda i: (i, 0))),
        out_shape=(jax.ShapeDtypeStruct((N, 1), jnp.float32),
                   jax.ShapeDtypeStruct((N, dout), jnp.float32)),
    )(degp, x, w)


def _tcb_mm_body(p_ref, u_ref, dinv_ref, b_ref, w_ref, out_ref, *, relu):
    z = dinv_ref[...] * (p_ref[0] + p_ref[1] + u_ref[...]) + b_ref[...]
    if relu:
        z = jnp.maximum(z, 0.0)
    out_ref[...] = dinv_ref[...] * jnp.dot(
        z, w_ref[...], preferred_element_type=jnp.float32)


def _tcb_ew_body(p_ref, u_ref, dinv_ref, b_ref, out_ref, *, relu):
    z = dinv_ref[...] * (p_ref[0] + p_ref[1] + u_ref[...]) + b_ref[...]
    if relu:
        z = jnp.maximum(z, 0.0)
    out_ref[...] = dinv_ref[...] * z


def _tcb(p, u, dinv, b, w, relu):
    din = u.shape[1]
    if w is None:
        return pl.pallas_call(
            functools.partial(_tcb_ew_body, relu=relu),
            grid=(_G,),
            in_specs=[
                pl.BlockSpec((NC, _B, din), lambda i: (0, i, 0)),
                pl.BlockSpec((_B, din), lambda i: (i, 0)),
                pl.BlockSpec((_B, 1), lambda i: (i, 0)),
                pl.BlockSpec((1, din), lambda i: (0, 0)),
            ],
            out_specs=pl.BlockSpec((_B, din), lambda i: (i, 0)),
            out_shape=jax.ShapeDtypeStruct((N, din), jnp.float32),
        )(p, u, dinv, b)
    dout = w.shape[1]
    return pl.pallas_call(
        functools.partial(_tcb_mm_body, relu=relu),
        grid=(_G,),
        in_specs=[
            pl.BlockSpec((NC, _B, din), lambda i: (0, i, 0)),
            pl.BlockSpec((_B, din), lambda i: (i, 0)),
            pl.BlockSpec((_B, 1), lambda i: (i, 0)),
            pl.BlockSpec((1, din), lambda i: (0, 0)),
            pl.BlockSpec((din, dout), lambda i: (0, 0)),
        ],
        out_specs=pl.BlockSpec((_B, dout), lambda i: (i, 0)),
        out_shape=jax.ShapeDtypeStruct((N, dout), jnp.float32),
    )(p, u, dinv, b, w)


def _pool_body(p_ref, u_ref, dinv_ref, wsg_ref, bsg_ref, gmax_ref):
    i = pl.program_id(0)
    h = dinv_ref[...] * (p_ref[0] + p_ref[1] + u_ref[...])
    z = jnp.dot(h, wsg_ref[...], preferred_element_type=jnp.float32)
    z = z + bsg_ref[...]
    m = jnp.max(z, axis=0, keepdims=True)

    @pl.when(i == 0)
    def _():
        gmax_ref[...] = m

    @pl.when(i > 0)
    def _():
        gmax_ref[...] = jnp.maximum(gmax_ref[...], m)


def _pool(p, u, dinv, wsg, bsg):
    return pl.pallas_call(
        _pool_body,
        grid=(_G,),
        in_specs=[
            pl.BlockSpec((NC, _B, LAT), lambda i: (0, i, 0)),
            pl.BlockSpec((_B, LAT), lambda i: (i, 0)),
            pl.BlockSpec((_B, 1), lambda i: (i, 0)),
            pl.BlockSpec((LAT, LAT), lambda i: (0, 0)),
            pl.BlockSpec((1, LAT), lambda i: (0, 0)),
        ],
        out_specs=pl.BlockSpec((1, LAT), lambda i: (0, 0)),
        out_shape=jax.ShapeDtypeStruct((1, LAT), jnp.float32),
    )(p, u, dinv, wsg, bsg)


def _head_body(g_ref, wf1, bf1, wf2, bf2, wc, bc, wo, bo, cpd_ref, comb_ref):
    g = jnp.maximum(jnp.dot(g_ref[...], wf1[...],
                            preferred_element_type=jnp.float32) + bf1[...], 0.)
    g = jnp.maximum(jnp.dot(g, wf2[...],
                            preferred_element_type=jnp.float32) + bf2[...], 0.)
    cpd_ref[...] = jnp.dot(g, wc[...],
                           preferred_element_type=jnp.float32) + bc[...]
    comb_ref[...] = jnp.dot(g, wo[...],
                            preferred_element_type=jnp.float32) + bo[...]


def _head(g, wf1, bf1, wf2, bf2, wc, bc, wo, bo):
    return pl.pallas_call(
        _head_body,
        out_shape=(jax.ShapeDtypeStruct((1, 1), jnp.float32),
                   jax.ShapeDtypeStruct((1, 1), jnp.float32)),
    )(g, wf1, bf1, wf2, bf2, wc, bc, wo, bo)


def kernel(x, edge_index, batch, W1, b1, W2, b2, Wg1, bg1, Wg2, bg2, Wg3, bg3,
           Wsg, bsg, Wf1, bf1, Wf2, bf2, Wc, bc, Wo, bo):
    del batch
    ei = edge_index.astype(jnp.int32)
    src3 = ei[0].reshape(NW, C, K)
    dst3 = ei[1].reshape(NW, C, K)
    srcw = ei[0].reshape(NW, 200, 50)
    dstw = ei[1].reshape(NW, 200, 50)

    ones_col = jnp.ones((K, _DW), jnp.float32)
    zeros_col = jnp.zeros((_NZT, _DW), jnp.float32)
    degp = _deg_call(dst3, ones_col, zeros_col)

    dinv, u = _tca(degp, x, W1)
    p = _prop128(u, srcw, dstw)
    u = _tcb(p, u, dinv, b1.reshape(1, -1), W2, relu=True)
    p = _prop64(u, src3, dst3)
    u = _tcb(p, u, dinv, b2.reshape(1, -1), Wg1, relu=False)
    p = _prop64(u, src3, dst3)
    u = _tcb(p, u, dinv, bg1.reshape(1, -1), Wg2, relu=True)
    p = _prop64(u, src3, dst3)
    u = _tcb(p, u, dinv, bg2.reshape(1, -1), Wg3, relu=True)
    p = _prop64(u, src3, dst3)
    u = _tcb(p, u, dinv, bg3.reshape(1, -1), None, relu=True)
    p = _prop64(u, src3, dst3)
    zb = jnp.zeros((1, LAT), jnp.float32)
    u = _tcb(p, u, dinv, zb, None, relu=False)
    p = _prop64(u, src3, dst3)
    u = _tcb(p, u, dinv, zb, None, relu=False)
    p = _prop64(u, src3, dst3)
    u = _tcb(p, u, dinv, zb, None, relu=False)
    p = _prop64(u, src3, dst3)

    g = _pool(p, u, dinv, Wsg, bsg.reshape(1, -1))
    cpd, comb = _head(g, Wf1, bf1.reshape(1, -1), Wf2, bf2.reshape(1, -1),
                      Wc, bc.reshape(1, -1), Wo, bo.reshape(1, -1))
    return (cpd, comb)


# speedup vs baseline: 29.9537x; 1.0326x over previous
import functools

import jax
import jax.numpy as jnp
from jax import lax
from jax.experimental import pallas as pl
from jax.experimental.pallas import tpu as pltpu
from jax.experimental.pallas import tpu_sc as plsc

N = 10000
E = 320000
D_IN = 128
LAT = 64

NC = 2
NS = 16
NW = NC * NS
EPW = E // NW
K = 125
C = EPW // K
KD = 16
CD = E // NS // KD
N2 = 10240
NPW = N2 // NW
NPT = N // NS
ZR = 125

_MESH = dict(core_axis_name="c", subcore_axis_name="s", num_cores=NC,
             num_subcores=NS)
_SC_PARAMS = pltpu.CompilerParams(use_tc_tiling_on_sc=False,
                                  needs_layout_passes=False)


_NZT = N2 // NS
_DW = 8


def _deg_body(dst_hbm, ones_hbm, zeros_hbm, degp_hbm, dstv, onesb, zbuf, acc1,
              sem):
    c = lax.axis_index("c")
    s = lax.axis_index("s")
    w = c * NS + s
    pltpu.sync_copy(dst_hbm.at[w], dstv)
    pltpu.sync_copy(ones_hbm, onesb)
    pltpu.sync_copy(zeros_hbm, zbuf)
    pltpu.sync_copy(zbuf, acc1.at[pl.ds(s * _NZT, _NZT)])
    plsc.subcore_barrier()

    def wave(j, _):
        pltpu.sync_copy(onesb, acc1.at[dstv.at[j]], add=True)
        return 0
    lax.fori_loop(0, C, wave, 0)
    plsc.subcore_barrier()
    pltpu.sync_copy(acc1.at[pl.ds(s * _NZT, _NZT)],
                    degp_hbm.at[c, pl.ds(s * _NZT, _NZT)])


_deg_call = functools.partial(
    pl.kernel, _deg_body,
    out_type=jax.ShapeDtypeStruct((NC, N2, _DW), jnp.float32),
    mesh=plsc.VectorSubcoreMesh(**_MESH),
    compiler_params=_SC_PARAMS,
    scratch_types=[
        pltpu.VMEM((C, K), jnp.int32),
        pltpu.VMEM((K, _DW), jnp.float32),
        pltpu.VMEM((_NZT, _DW), jnp.float32),
        pltpu.VMEM_SHARED((N2, _DW), jnp.float32),
        pltpu.SemaphoreType.DMA,
    ],
)()


def _prop_body(u_hbm, src_hbm, dst_hbm, p_hbm, *rest, d, k, c_, nb):
    srcv, dstv = rest[0], rest[1]
    gbufs = rest[2:2 + nb]
    gsems = rest[2 + nb:2 + 2 * nb]
    ssems = rest[2 + 2 * nb:2 + 3 * nb]
    acc = rest[2 + 3 * nb]
    c = lax.axis_index("c")
    s = lax.axis_index("s")
    w = c * NS + s
    pltpu.sync_copy(src_hbm.at[w], srcv)
    pltpu.sync_copy(dst_hbm.at[w], dstv)
    zero = jnp.zeros((16,), jnp.float32)

    def zb(i, _):
        for t in range(d // 16):
            gbufs[0][i, pl.ds(t * 16, 16)] = zero
        return 0
    lax.fori_loop(0, k, zb, 0)
    nfull, rem = divmod(NPT, k)
    for t in range(nfull):
        pltpu.sync_copy(gbufs[0], acc.at[pl.ds(s * NPT + t * k, k)])
    if rem:
        pltpu.sync_copy(gbufs[0].at[pl.ds(0, rem)],
                        acc.at[pl.ds(s * NPT + nfull * k, rem)])
    for t in range(nb):
        pltpu.async_copy(u_hbm.at[srcv.at[t]], gbufs[t], gsems[t])
    plsc.subcore_barrier()

    def ring(i, _):
        base = i * nb
        for t in range(nb):
            j = base + t
            pltpu.make_async_copy(u_hbm.at[srcv.at[j]], gbufs[t],
                                  gsems[t]).wait()
            pltpu.async_copy(gbufs[t], acc.at[dstv.at[j]], ssems[t],
                             add=True)
        for t in range(nb):
            j = base + t
            pltpu.make_async_copy(gbufs[t], acc.at[dstv.at[j]],
                                  ssems[t]).wait()

            @pl.when(j + nb < c_)
            def _():
                pltpu.async_copy(u_hbm.at[srcv.at[j + nb]], gbufs[t],
                                 gsems[t])
        return 0
    lax.fori_loop(0, c_ // nb, ring, 0)
    plsc.subcore_barrier()
    pltpu.sync_copy(acc.at[pl.ds(s * NPT, NPT)],
                    p_hbm.at[c, pl.ds(s * NPT, NPT)])


def _make_prop(d, k, nb):
    c_ = EPW // k
    assert c_ % nb == 0
    return functools.partial(
        pl.kernel, functools.partial(_prop_body, d=d, k=k, c_=c_, nb=nb),
        out_type=jax.ShapeDtypeStruct((NC, N, d), jnp.float32),
        mesh=plsc.VectorSubcoreMesh(**_MESH),
        compiler_params=_SC_PARAMS,
        scratch_types=(
            [pltpu.VMEM((c_, k), jnp.int32),
             pltpu.VMEM((c_, k), jnp.int32)]
            + [pltpu.VMEM((k, d), jnp.float32)] * nb
            + [pltpu.SemaphoreType.DMA] * (2 * nb)
            + [pltpu.VMEM_SHARED((N, d), jnp.float32)]
        ),
    )()


_prop128 = _make_prop(D_IN, 50, 4)
_prop64 = _make_prop(LAT, K, 8)

_B = 2000
_G = N // _B


def _tca_body(degp_ref, x_ref, w_ref, dinv_ref, u_ref):
    deg = degp_ref[0, :, 0:1] + degp_ref[1, :, 0:1] + 1.0
    dinv = jax.lax.rsqrt(deg)
    dinv_ref[...] = dinv
    u_ref[...] = dinv * jnp.dot(
        x_ref[...], w_ref[...], preferred_element_type=jnp.float32)


def _tca(degp, x, w):
    din, dout = w.shape
    return pl.pallas_call(
        _tca_body,
        grid=(_G,),
        in_specs=[
            pl.BlockSpec((NC, _B, _DW), lambda i: (0, i, 0)),
            pl.BlockSpec((_B, din), lambda i: (i, 0)),
            pl.BlockSpec((din, dout), lambda i: (0, 0)),
        ],
        out_specs=(pl.BlockSpec((_B, 1), lambda i: (i, 0)),
                   pl.BlockSpec((_B, dout), lambda i: (i, 0))),
        out_shape=(jax.ShapeDtypeStruct((N, 1), jnp.float32),
                   jax.ShapeDtypeStruct((N, dout), jnp.float32)),
    )(degp, x, w)


def _tcb_mm_body(p_ref, u_ref, dinv_ref, b_ref, w_ref, out_ref, *, relu):
    z = dinv_ref[...] * (p_ref[0] + p_ref[1] + u_ref[...]) + b_ref[...]
    if relu:
        z = jnp.maximum(z, 0.0)
    out_ref[...] = dinv_ref[...] * jnp.dot(
        z, w_ref[...], preferred_element_type=jnp.float32)


def _tcb_ew_body(p_ref, u_ref, dinv_ref, b_ref, out_ref, *, relu):
    z = dinv_ref[...] * (p_ref[0] + p_ref[1] + u_ref[...]) + b_ref[...]
    if relu:
        z = jnp.maximum(z, 0.0)
    out_ref[...] = dinv_ref[...] * z


def _tcb(p, u, dinv, b, w, relu):
    din = u.shape[1]
    if w is None:
        return pl.pallas_call(
            functools.partial(_tcb_ew_body, relu=relu),
            grid=(_G,),
            in_specs=[
                pl.BlockSpec((NC, _B, din), lambda i: (0, i, 0)),
                pl.BlockSpec((_B, din), lambda i: (i, 0)),
                pl.BlockSpec((_B, 1), lambda i: (i, 0)),
                pl.BlockSpec((1, din), lambda i: (0, 0)),
            ],
            out_specs=pl.BlockSpec((_B, din), lambda i: (i, 0)),
            out_shape=jax.ShapeDtypeStruct((N, din), jnp.float32),
        )(p, u, dinv, b)
    dout = w.shape[1]
    return pl.pallas_call(
        functools.partial(_tcb_mm_body, relu=relu),
        grid=(_G,),
        in_specs=[
            pl.BlockSpec((NC, _B, din), lambda i: (0, i, 0)),
            pl.BlockSpec((_B, din), lambda i: (i, 0)),
            pl.BlockSpec((_B, 1), lambda i: (i, 0)),
            pl.BlockSpec((1, din), lambda i: (0, 0)),
            pl.BlockSpec((din, dout), lambda i: (0, 0)),
        ],
        out_specs=pl.BlockSpec((_B, dout), lambda i: (i, 0)),
        out_shape=jax.ShapeDtypeStruct((N, dout), jnp.float32),
    )(p, u, dinv, b, w)


def _pool_body(p_ref, u_ref, dinv_ref, wsg_ref, bsg_ref, gmax_ref):
    i = pl.program_id(0)
    h = dinv_ref[...] * (p_ref[0] + p_ref[1] + u_ref[...])
    z = jnp.dot(h, wsg_ref[...], preferred_element_type=jnp.float32)
    z = z + bsg_ref[...]
    m = jnp.max(z, axis=0, keepdims=True)

    @pl.when(i == 0)
    def _():
        gmax_ref[...] = m

    @pl.when(i > 0)
    def _():
        gmax_ref[...] = jnp.maximum(gmax_ref[...], m)


def _pool(p, u, dinv, wsg, bsg):
    return pl.pallas_call(
        _pool_body,
        grid=(_G,),
        in_specs=[
            pl.BlockSpec((NC, _B, LAT), lambda i: (0, i, 0)),
            pl.BlockSpec((_B, LAT), lambda i: (i, 0)),
            pl.BlockSpec((_B, 1), lambda i: (i, 0)),
            pl.BlockSpec((LAT, LAT), lambda i: (0, 0)),
            pl.BlockSpec((1, LAT), lambda i: (0, 0)),
        ],
        out_specs=pl.BlockSpec((1, LAT), lambda i: (0, 0)),
        out_shape=jax.ShapeDtypeStruct((1, LAT), jnp.float32),
    )(p, u, dinv, wsg, bsg)


def _head_body(g_ref, wf1, bf1, wf2, bf2, wc, bc, wo, bo, cpd_ref, comb_ref):
    g = jnp.maximum(jnp.dot(g_ref[...], wf1[...],
                            preferred_element_type=jnp.float32) + bf1[...], 0.)
    g = jnp.maximum(jnp.dot(g, wf2[...],
                            preferred_element_type=jnp.float32) + bf2[...], 0.)
    cpd_ref[...] = jnp.dot(g, wc[...],
                           preferred_element_type=jnp.float32) + bc[...]
    comb_ref[...] = jnp.dot(g, wo[...],
                            preferred_element_type=jnp.float32) + bo[...]


def _head(g, wf1, bf1, wf2, bf2, wc, bc, wo, bo):
    return pl.pallas_call(
        _head_body,
        out_shape=(jax.ShapeDtypeStruct((1, 1), jnp.float32),
                   jax.ShapeDtypeStruct((1, 1), jnp.float32)),
    )(g, wf1, bf1, wf2, bf2, wc, bc, wo, bo)


def kernel(x, edge_index, batch, W1, b1, W2, b2, Wg1, bg1, Wg2, bg2, Wg3, bg3,
           Wsg, bsg, Wf1, bf1, Wf2, bf2, Wc, bc, Wo, bo):
    del batch
    ei = edge_index.astype(jnp.int32)
    src3 = ei[0].reshape(NW, C, K)
    dst3 = ei[1].reshape(NW, C, K)
    srcw = ei[0].reshape(NW, 200, 50)
    dstw = ei[1].reshape(NW, 200, 50)

    ones_col = jnp.ones((K, _DW), jnp.float32)
    zeros_col = jnp.zeros((_NZT, _DW), jnp.float32)
    degp = _deg_call(dst3, ones_col, zeros_col)

    dinv, u = _tca(degp, x, W1)
    p = _prop128(u, srcw, dstw)
    u = _tcb(p, u, dinv, b1.reshape(1, -1), W2, relu=True)
    p = _prop64(u, src3, dst3)
    u = _tcb(p, u, dinv, b2.reshape(1, -1), Wg1, relu=False)
    p = _prop64(u, src3, dst3)
    u = _tcb(p, u, dinv, bg1.reshape(1, -1), Wg2, relu=True)
    p = _prop64(u, src3, dst3)
    u = _tcb(p, u, dinv, bg2.reshape(1, -1), Wg3, relu=True)
    p = _prop64(u, src3, dst3)
    u = _tcb(p, u, dinv, bg3.reshape(1, -1), None, relu=True)
    p = _prop64(u, src3, dst3)
    zb = jnp.zeros((1, LAT), jnp.float32)
    u = _tcb(p, u, dinv, zb, None, relu=False)
    p = _prop64(u, src3, dst3)
    u = _tcb(p, u, dinv, zb, None, relu=False)
    p = _prop64(u, src3, dst3)
    u = _tcb(p, u, dinv, zb, None, relu=False)
    p = _prop64(u, src3, dst3)

    g = _pool(p, u, dinv, Wsg, bsg.reshape(1, -1))
    cpd, comb = _head(g, Wf1, bf1.reshape(1, -1), Wf2, bf2.reshape(1, -1),
                      Wc, bc.reshape(1, -1), Wo, bo.reshape(1, -1))
    return (cpd, comb)
